# baseline (speedup 1.0000x reference)
import functools

import jax
import jax.numpy as jnp
from jax import lax
from jax.experimental import pallas as pl
from jax.experimental.pallas import tpu as pltpu
from jax.experimental.pallas import tpu_sc as plsc

_B = 16384
_D = 64
_NC = 2
_NS = 16
_NW = _NC * _NS
_PW = _B // _NW
_H = 4
_QB = _PW // _H
_C = 128

_BC = 1024


def _pool_quarter(idbuf, tabv, sumv, q, slots):
  i32 = jnp.int32
  f32 = jnp.float32
  iota = lax.iota(i32, 16)

  def body(g, carry):
    sb = q * _QB + g * 16
    accs = [jnp.zeros((16,), f32) for _ in range(16)]
    for j in range(slots):
      pvec = iota * slots + (sb * slots + j)
      tv = plsc.load_gather(idbuf, [pvec])
      mf = (tv != 0).astype(f32)
      fbase = tv * 16
      for d in range(16):
        vals = plsc.load_gather(tabv, [fbase + d])
        accs[d] = accs[d] + vals * mf
    rowv = g * 16 + iota
    for d in range(16):
      plsc.store_scatter(sumv, [rowv, jnp.full((16,), d, i32)],
                         accs[d])
    return carry

  lax.fori_loop(0, _QB // 16, body, 0)


def _lookup_quarter(idbuf, tabv, sumv, q):
  i32 = jnp.int32
  iota = lax.iota(i32, 16)

  def body(g, carry):
    sb = q * _QB + g * 16
    cv = idbuf[pl.ds(sb, 16)]
    fbase = cv * 16
    rowv = g * 16 + iota
    for d in range(16):
      vals = plsc.load_gather(tabv, [fbase + d])
      plsc.store_scatter(sumv, [rowv, jnp.full((16,), d, i32)],
                         vals)
    return carry

  lax.fori_loop(0, _QB // 16, body, 0)


def _sc_body(u128, uid, d128, did, s128, sid, ttab, tagid, tstab, tasteid,
             ctab, catid,
             urows, drows, srows, tagsums, tastesums, catrows,
             tagtab_v, tastetab_v, cattab_v, it1, its1, ict1,
             iu, idh, ist, ru, rd, rs, tsum, tssum, csum,
             sT, sI, sG, sO):
  wid = lax.axis_index("s") * _NC + lax.axis_index("c")
  b0 = wid * _PW
  pre = [
      pltpu.async_copy(ttab, tagtab_v, sT),
      pltpu.async_copy(tstab, tastetab_v, sT),
      pltpu.async_copy(ctab, cattab_v, sT),
      pltpu.async_copy(tagid.at[pl.ds(b0 * 5, _PW * 5)], it1, sT),
      pltpu.async_copy(tasteid.at[pl.ds(b0 * 3, _PW * 3)], its1, sT),
      pltpu.async_copy(catid.at[pl.ds(b0, _PW)], ict1, sT),
  ]
  for cp in pre:
    cp.wait()
  for q in range(_H):
    b = b0 + q * _QB
    ics = [
        pltpu.async_copy(uid.at[pl.ds(b, _QB)], iu.at[0], sI),
        pltpu.async_copy(did.at[pl.ds(b, _QB)], idh.at[0], sI),
        pltpu.async_copy(sid.at[pl.ds(b, _QB)], ist.at[0], sI),
    ]
    for cp in ics:
      cp.wait()
    for k in range(_C // 16):
      sl = pl.ds(k * 16, 16)
      iu[0, sl] = lax.shift_right_logical(iu[0, sl], 1)
      idh[0, sl] = lax.shift_right_logical(idh[0, sl], 1)
      ist[0, sl] = lax.shift_right_logical(ist[0, sl], 2)
    gcs = [
        pltpu.async_copy(u128.at[iu.at[0]], ru, sG),
        pltpu.async_copy(d128.at[idh.at[0]], rd, sG),
        pltpu.async_copy(s128.at[ist.at[0]], rs, sG),
    ]
    _pool_quarter(it1, tagtab_v, tsum, q, 5)
    _pool_quarter(its1, tastetab_v, tssum, q, 3)
    _lookup_quarter(ict1, cattab_v, csum, q)
    for cp in gcs:
      cp.wait()
    ocs = [
        pltpu.async_copy(ru, urows.at[pl.ds(b, _QB)], sO),
        pltpu.async_copy(rd, drows.at[pl.ds(b, _QB)], sO),
        pltpu.async_copy(rs, srows.at[pl.ds(b, _QB)], sO),
        pltpu.async_copy(tsum, tagsums.at[pl.ds(b, _QB)], sO),
        pltpu.async_copy(tssum, tastesums.at[pl.ds(b, _QB)], sO),
        pltpu.async_copy(csum, catrows.at[pl.ds(b, _QB)], sO),
    ]
    for cp in ocs:
      cp.wait()


@functools.cache
def _sc_gather_kernel():
  f32 = jnp.float32
  i32 = jnp.int32
  return pl.kernel(
      _sc_body,
      out_type=[
          jax.ShapeDtypeStruct((_B, 128), f32),
          jax.ShapeDtypeStruct((_B, 128), f32),
          jax.ShapeDtypeStruct((_B, 128), f32),
          jax.ShapeDtypeStruct((_B, 16), f32),
          jax.ShapeDtypeStruct((_B, 16), f32),
          jax.ShapeDtypeStruct((_B, 16), f32),
      ],
      mesh=plsc.VectorSubcoreMesh(core_axis_name="c", subcore_axis_name="s",
                                  num_cores=_NC, num_subcores=_NS),
      scratch_types=[
          pltpu.VMEM((16000,), f32),
          pltpu.VMEM((16000,), f32),
          pltpu.VMEM((16000,), f32),
          pltpu.VMEM((_PW * 5,), i32),
          pltpu.VMEM((_PW * 3,), i32),
          pltpu.VMEM((_PW,), i32),
          pltpu.VMEM((1, _C), i32),
          pltpu.VMEM((1, _C), i32),
          pltpu.VMEM((1, _C), i32),
          pltpu.VMEM((_QB, 128), f32),
          pltpu.VMEM((_QB, 128), f32),
          pltpu.VMEM((_QB, 128), f32),
          pltpu.VMEM((_QB, 16), f32),
          pltpu.VMEM((_QB, 16), f32),
          pltpu.VMEM((_QB, 16), f32),
          pltpu.SemaphoreType.DMA,
          pltpu.SemaphoreType.DMA,
          pltpu.SemaphoreType.DMA,
          pltpu.SemaphoreType.DMA,
      ],
      compiler_params=pltpu.CompilerParams(use_tc_tiling_on_sc=False,
                                           needs_layout_passes=False),
  )


def _tc_body(urows, drows, srows, tagsums, tastesums, catrows,
             uidc, didc, sidc,
             age, gender, uloc, utime, uday, rec,
             tags, tastes, price, order, rating, iloc, itime, iday,
             age_W, age_b, gender_tab, uloc_W, uloc_b, utime_W, utime_b,
             uday_tab, rec_W, rec_b,
             price_W, price_b, order_W, order_b, rating_W, rating_b,
             iloc_W, iloc_b, itime_W, itime_b, iday_tab,
             up_W, up_b, ip_W, ip_b,
             un_out, it_out, sc_out):
  f32 = jnp.float32
  Wu = up_W[...]
  Wi = ip_W[...]

  ur = urows[...]
  pe = (lax.rem(uidc[...], 2) == 0).astype(f32)
  uemb = ur[:, 0:64] * pe + ur[:, 64:128] * (1.0 - pe)
  dr = drows[...]
  de = (lax.rem(didc[...], 2) == 0).astype(f32)
  demb = dr[:, 0:64] * de + dr[:, 64:128] * (1.0 - de)
  sr = srows[...]
  sp = lax.rem(sidc[...], 4)
  semb = (sr[:, 0:32] * (sp == 0).astype(f32)
          + sr[:, 32:64] * (sp == 1).astype(f32)
          + sr[:, 64:96] * (sp == 2).astype(f32)
          + sr[:, 96:128] * (sp == 3).astype(f32))

  uv = jnp.dot(uemb, Wu[0:64], preferred_element_type=f32)
  uv += age[...] * jnp.dot(age_W[...], Wu[64:80], preferred_element_type=f32)
  g1h = (gender[...] == lax.broadcasted_iota(jnp.int32, (_BC, 3), 1)).astype(f32)
  uv += jnp.dot(g1h, jnp.dot(gender_tab[...], Wu[80:96],
                             preferred_element_type=f32),
                preferred_element_type=f32)
  uv += jnp.dot(uloc[...], jnp.dot(uloc_W[...], Wu[96:112],
                                   preferred_element_type=f32),
                preferred_element_type=f32)
  uv += utime[...] * jnp.dot(utime_W[...], Wu[112:120], preferred_element_type=f32)
  ud1h = (uday[...] == lax.broadcasted_iota(jnp.int32, (_BC, 7), 1)).astype(f32)
  uv += jnp.dot(ud1h, jnp.dot(uday_tab[...], Wu[120:128],
                              preferred_element_type=f32),
                preferred_element_type=f32)
  uv += rec[...] * jnp.dot(rec_W[...], Wu[128:144], preferred_element_type=f32)
  ubias = (jnp.dot(age_b[...], Wu[64:80], preferred_element_type=f32)
           + jnp.dot(uloc_b[...], Wu[96:112], preferred_element_type=f32)
           + jnp.dot(utime_b[...], Wu[112:120], preferred_element_type=f32)
           + jnp.dot(rec_b[...], Wu[128:144], preferred_element_type=f32)
           + up_b[...])
  uv += ubias

  iv = jnp.dot(demb, Wi[0:64], preferred_element_type=f32)
  iv += jnp.dot(semb, Wi[64:96], preferred_element_type=f32)
  m_tag = (tags[...] != 0).astype(f32)
  inv_t = 1.0 / (jnp.sum(m_tag, axis=1, keepdims=True) + 1e-08)
  iv += jnp.dot(tagsums[...] * inv_t, Wi[96:112], preferred_element_type=f32)
  m_ts = (tastes[...] != 0).astype(f32)
  inv_s = 1.0 / (jnp.sum(m_ts, axis=1, keepdims=True) + 1e-08)
  iv += jnp.dot(tastesums[...] * inv_s, Wi[112:128], preferred_element_type=f32)
  iv += jnp.dot(catrows[...], Wi[128:144], preferred_element_type=f32)
  iv += price[...] * jnp.dot(price_W[...], Wi[144:160], preferred_element_type=f32)
  iv += order[...] * jnp.dot(order_W[...], Wi[160:168], preferred_element_type=f32)
  iv += rating[...] * jnp.dot(rating_W[...], Wi[168:176], preferred_element_type=f32)
  iv += jnp.dot(iloc[...], jnp.dot(iloc_W[...], Wi[176:192],
                                   preferred_element_type=f32),
                preferred_element_type=f32)
  iv += itime[...] * jnp.dot(itime_W[...], Wi[192:200], preferred_element_type=f32)
  id1h = (iday[...] == lax.broadcasted_iota(jnp.int32, (_BC, 7), 1)).astype(f32)
  iv += jnp.dot(id1h, jnp.dot(iday_tab[...], Wi[200:208],
                              preferred_element_type=f32),
                preferred_element_type=f32)
  ibias = (jnp.dot(price_b[...], Wi[144:160], preferred_element_type=f32)
           + jnp.dot(order_b[...], Wi[160:168], preferred_element_type=f32)
           + jnp.dot(rating_b[...], Wi[168:176], preferred_element_type=f32)
           + jnp.dot(iloc_b[...], Wi[176:192], preferred_element_type=f32)
           + jnp.dot(itime_b[...], Wi[192:200], preferred_element_type=f32)
           + ip_b[...])
  iv += ibias

  un = uv / jnp.maximum(jnp.sqrt(jnp.sum(uv * uv, axis=-1, keepdims=True)), 1e-12)
  it = iv / jnp.maximum(jnp.sqrt(jnp.sum(iv * iv, axis=-1, keepdims=True)), 1e-12)
  un_out[...] = un
  it_out[...] = it
  sc_out[...] = jnp.sum(un * it, axis=-1, keepdims=True)


def _chunk(d):
  return pl.BlockSpec((_BC, d), lambda i: (i, 0))


def _full(shape):
  return pl.BlockSpec(shape, lambda i: (0,) * len(shape))


def kernel(user_id, age, gender, user_location, user_time_of_day,
           user_day_of_week, recency, dish_id, store_id, tags, tastes,
           category, price, order_times, rating, item_location,
           item_time_of_day, item_day_of_week, user_emb_table, user_age_W,
           user_age_b, user_gender_table, user_location_W, user_location_b,
           user_time_W, user_time_b, user_day_table, user_recency_W,
           user_recency_b, dish_emb_table, store_emb_table, tag_emb_table,
           taste_emb_table, cat_emb_table, dish_price_W, dish_price_b,
           dish_order_times_W, dish_order_times_b, dish_rating_W,
           dish_rating_b, dish_location_W, dish_location_b, dish_time_W,
           dish_time_b, dish_day_table, user_proj_W, user_proj_b,
           item_proj_W, item_proj_b):
  i32 = jnp.int32
  uid1 = user_id.astype(i32)
  did1 = dish_id.astype(i32)
  sid1 = store_id.astype(i32)
  tag1 = tags.astype(i32).reshape(_B * 5)
  tas1 = tastes.astype(i32).reshape(_B * 3)
  cat1 = category.astype(i32)

  u128 = user_emb_table.reshape(-1, 128)
  d128 = dish_emb_table.reshape(-1, 128)
  s128 = store_emb_table.reshape(-1, 128)
  t1 = tag_emb_table.reshape(-1)
  ts1 = taste_emb_table.reshape(-1)
  c1 = cat_emb_table.reshape(-1)

  urows, drows, srows, tagsums, tastesums, catrows = _sc_gather_kernel()(
      u128, uid1, d128, did1, s128, sid1, t1, tag1, ts1, tas1, c1, cat1)

  grid = (_B // _BC,)
  un, it, sc = pl.pallas_call(
      _tc_body,
      grid=grid,
      in_specs=[
          _chunk(128), _chunk(128), _chunk(128), _chunk(16), _chunk(16),
          _chunk(16),
          _chunk(1), _chunk(1), _chunk(1),
          _chunk(1), _chunk(1), _chunk(2), _chunk(1), _chunk(1), _chunk(1),
          _chunk(5), _chunk(3), _chunk(1), _chunk(1), _chunk(1), _chunk(2),
          _chunk(1), _chunk(1),
          _full((1, 16)), _full((1, 16)), _full((3, 16)), _full((2, 16)),
          _full((1, 16)), _full((1, 8)), _full((1, 8)), _full((7, 8)),
          _full((1, 16)), _full((1, 16)),
          _full((1, 16)), _full((1, 16)), _full((1, 8)), _full((1, 8)),
          _full((1, 8)), _full((1, 8)), _full((2, 16)), _full((1, 16)),
          _full((1, 8)), _full((1, 8)), _full((7, 8)),
          _full((144, 64)), _full((1, 64)), _full((208, 64)), _full((1, 64)),
      ],
      out_specs=[_chunk(64), _chunk(64), _chunk(1)],
      out_shape=[
          jax.ShapeDtypeStruct((_B, 64), jnp.float32),
          jax.ShapeDtypeStruct((_B, 64), jnp.float32),
          jax.ShapeDtypeStruct((_B, 1), jnp.float32),
      ],
  )(
      urows, drows, srows, tagsums, tastesums, catrows,
      uid1.reshape(_B, 1), did1.reshape(_B, 1), sid1.reshape(_B, 1),
      age.reshape(_B, 1), gender.astype(i32).reshape(_B, 1), user_location,
      user_time_of_day.reshape(_B, 1),
      user_day_of_week.astype(i32).reshape(_B, 1), recency.reshape(_B, 1),
      tags.astype(i32), tastes.astype(i32), price.reshape(_B, 1),
      order_times.reshape(_B, 1), rating.reshape(_B, 1), item_location,
      item_time_of_day.reshape(_B, 1),
      item_day_of_week.astype(i32).reshape(_B, 1),
      user_age_W, user_age_b.reshape(1, 16), user_gender_table,
      user_location_W, user_location_b.reshape(1, 16), user_time_W,
      user_time_b.reshape(1, 8), user_day_table, user_recency_W,
      user_recency_b.reshape(1, 16),
      dish_price_W, dish_price_b.reshape(1, 16), dish_order_times_W,
      dish_order_times_b.reshape(1, 8), dish_rating_W,
      dish_rating_b.reshape(1, 8), dish_location_W,
      dish_location_b.reshape(1, 16), dish_time_W, dish_time_b.reshape(1, 8),
      dish_day_table,
      user_proj_W, user_proj_b.reshape(1, 64), item_proj_W,
      item_proj_b.reshape(1, 64),
  )
  return (un, it, sc.reshape(_B))


# speedup vs baseline: 1.0190x; 1.0190x over previous
import functools

import jax
import jax.numpy as jnp
from jax import lax
from jax.experimental import pallas as pl
from jax.experimental.pallas import tpu as pltpu
from jax.experimental.pallas import tpu_sc as plsc

_B = 16384
_D = 64
_NC = 2
_NS = 16
_NW = _NC * _NS
_PW = _B // _NW
_H = 8
_QB = _PW // _H
_C = 128

_BC = 1024


def _pool_chunk(idbuf, tabv, sumv, q, slots, dofs):
  i32 = jnp.int32
  f32 = jnp.float32
  iota = lax.iota(i32, 16)

  def body(g, carry):
    sb = q * _QB + g * 16
    accs = [jnp.zeros((16,), f32) for _ in range(16)]
    for j in range(slots):
      pvec = iota * slots + (sb * slots + j)
      tv = plsc.load_gather(idbuf, [pvec])
      mf = (tv != 0).astype(f32)
      fbase = tv * 16
      for d in range(16):
        vals = plsc.load_gather(tabv, [fbase + d])
        accs[d] = accs[d] + vals * mf
    rowv = g * 16 + iota
    for d in range(16):
      plsc.store_scatter(sumv, [rowv, jnp.full((16,), dofs + d, i32)],
                         accs[d])
    return carry

  lax.fori_loop(0, _QB // 16, body, 0)


def _lookup_chunk(idbuf, tabv, sumv, q, dofs):
  i32 = jnp.int32
  iota = lax.iota(i32, 16)

  def body(g, carry):
    sb = q * _QB + g * 16
    cv = idbuf[pl.ds(sb, 16)]
    fbase = cv * 16
    rowv = g * 16 + iota
    for d in range(16):
      vals = plsc.load_gather(tabv, [fbase + d])
      plsc.store_scatter(sumv, [rowv, jnp.full((16,), dofs + d, i32)],
                         vals)
    return carry

  lax.fori_loop(0, _QB // 16, body, 0)


def _sc_body(u128, uid, d128, did, s128, sid, ttab, tagid, tstab, tasteid,
             ctab, catid,
             urows, drows, srows, pools,
             tagtab_v, tastetab_v, cattab_v, it1, its1, ict1,
             iu, idh, ist, ru, rd, rs, psum,
             sT, sI, sG, sO):
  wid = lax.axis_index("s") * _NC + lax.axis_index("c")
  b0 = wid * _PW
  pre = [
      pltpu.async_copy(ttab, tagtab_v, sT),
      pltpu.async_copy(tstab, tastetab_v, sT),
      pltpu.async_copy(ctab, cattab_v, sT),
      pltpu.async_copy(tagid.at[pl.ds(b0 * 5, _PW * 5)], it1, sT),
      pltpu.async_copy(tasteid.at[pl.ds(b0 * 3, _PW * 3)], its1, sT),
      pltpu.async_copy(catid.at[pl.ds(b0, _PW)], ict1, sT),
  ]
  for cp in pre:
    cp.wait()
  for q in range(_H):
    b = b0 + q * _QB
    ics = [
        pltpu.async_copy(uid.at[pl.ds(b, _QB)], iu.at[0], sI),
        pltpu.async_copy(did.at[pl.ds(b, _QB)], idh.at[0], sI),
        pltpu.async_copy(sid.at[pl.ds(b, _QB)], ist.at[0], sI),
    ]
    for cp in ics:
      cp.wait()
    for k in range(_QB // 16):
      sl = pl.ds(k * 16, 16)
      iu[0, sl] = lax.shift_right_logical(iu[0, sl], 1)
      idh[0, sl] = lax.shift_right_logical(idh[0, sl], 1)
      ist[0, sl] = lax.shift_right_logical(ist[0, sl], 2)
    gcs = [
        pltpu.async_copy(u128.at[iu.at[0]], ru, sG),
        pltpu.async_copy(d128.at[idh.at[0]], rd, sG),
        pltpu.async_copy(s128.at[ist.at[0]], rs, sG),
    ]
    _pool_chunk(it1, tagtab_v, psum, q, 5, 0)
    _pool_chunk(its1, tastetab_v, psum, q, 3, 16)
    _lookup_chunk(ict1, cattab_v, psum, q, 32)
    for cp in gcs:
      cp.wait()
    ocs = [
        pltpu.async_copy(ru, urows.at[pl.ds(b, _QB)], sO),
        pltpu.async_copy(rd, drows.at[pl.ds(b, _QB)], sO),
        pltpu.async_copy(rs, srows.at[pl.ds(b, _QB)], sO),
        pltpu.async_copy(psum, pools.at[pl.ds(b, _QB)], sO),
    ]
    for cp in ocs:
      cp.wait()


@functools.cache
def _sc_gather_kernel():
  f32 = jnp.float32
  i32 = jnp.int32
  return pl.kernel(
      _sc_body,
      out_type=[
          jax.ShapeDtypeStruct((_B, 128), f32),
          jax.ShapeDtypeStruct((_B, 128), f32),
          jax.ShapeDtypeStruct((_B, 128), f32),
          jax.ShapeDtypeStruct((_B, 128), f32),
      ],
      mesh=plsc.VectorSubcoreMesh(core_axis_name="c", subcore_axis_name="s",
                                  num_cores=_NC, num_subcores=_NS),
      scratch_types=[
          pltpu.VMEM((16000,), f32),
          pltpu.VMEM((16000,), f32),
          pltpu.VMEM((16000,), f32),
          pltpu.VMEM((_PW * 5,), i32),
          pltpu.VMEM((_PW * 3,), i32),
          pltpu.VMEM((_PW,), i32),
          pltpu.VMEM((1, _QB), i32),
          pltpu.VMEM((1, _QB), i32),
          pltpu.VMEM((1, _QB), i32),
          pltpu.VMEM((_QB, 128), f32),
          pltpu.VMEM((_QB, 128), f32),
          pltpu.VMEM((_QB, 128), f32),
          pltpu.VMEM((_QB, 128), f32),
          pltpu.SemaphoreType.DMA,
          pltpu.SemaphoreType.DMA,
          pltpu.SemaphoreType.DMA,
          pltpu.SemaphoreType.DMA,
      ],
      compiler_params=pltpu.CompilerParams(use_tc_tiling_on_sc=True,
                                           needs_layout_passes=False),
  )


def _tc_body(urows, drows, srows, pools,
             uidc, didc, sidc,
             age, gender, uloc, utime, uday, rec,
             tags, tastes, price, order, rating, iloc, itime, iday,
             age_W, age_b, gender_tab, uloc_W, uloc_b, utime_W, utime_b,
             uday_tab, rec_W, rec_b,
             price_W, price_b, order_W, order_b, rating_W, rating_b,
             iloc_W, iloc_b, itime_W, itime_b, iday_tab,
             up_W, up_b, ip_W, ip_b,
             un_out, it_out, sc_out):
  f32 = jnp.float32
  Wu = up_W[...]
  Wi = ip_W[...]

  ur = urows[...]
  pe = (lax.rem(uidc[...], 2) == 0).astype(f32)
  uemb = ur[:, 0:64] * pe + ur[:, 64:128] * (1.0 - pe)
  dr = drows[...]
  de = (lax.rem(didc[...], 2) == 0).astype(f32)
  demb = dr[:, 0:64] * de + dr[:, 64:128] * (1.0 - de)
  sr = srows[...]
  sp = lax.rem(sidc[...], 4)
  semb = (sr[:, 0:32] * (sp == 0).astype(f32)
          + sr[:, 32:64] * (sp == 1).astype(f32)
          + sr[:, 64:96] * (sp == 2).astype(f32)
          + sr[:, 96:128] * (sp == 3).astype(f32))

  uv = jnp.dot(uemb, Wu[0:64], preferred_element_type=f32)
  uv += age[...] * jnp.dot(age_W[...], Wu[64:80], preferred_element_type=f32)
  g1h = (gender[...] == lax.broadcasted_iota(jnp.int32, (_BC, 3), 1)).astype(f32)
  uv += jnp.dot(g1h, jnp.dot(gender_tab[...], Wu[80:96],
                             preferred_element_type=f32),
                preferred_element_type=f32)
  uv += jnp.dot(uloc[...], jnp.dot(uloc_W[...], Wu[96:112],
                                   preferred_element_type=f32),
                preferred_element_type=f32)
  uv += utime[...] * jnp.dot(utime_W[...], Wu[112:120], preferred_element_type=f32)
  ud1h = (uday[...] == lax.broadcasted_iota(jnp.int32, (_BC, 7), 1)).astype(f32)
  uv += jnp.dot(ud1h, jnp.dot(uday_tab[...], Wu[120:128],
                              preferred_element_type=f32),
                preferred_element_type=f32)
  uv += rec[...] * jnp.dot(rec_W[...], Wu[128:144], preferred_element_type=f32)
  ubias = (jnp.dot(age_b[...], Wu[64:80], preferred_element_type=f32)
           + jnp.dot(uloc_b[...], Wu[96:112], preferred_element_type=f32)
           + jnp.dot(utime_b[...], Wu[112:120], preferred_element_type=f32)
           + jnp.dot(rec_b[...], Wu[128:144], preferred_element_type=f32)
           + up_b[...])
  uv += ubias

  iv = jnp.dot(demb, Wi[0:64], preferred_element_type=f32)
  iv += jnp.dot(semb, Wi[64:96], preferred_element_type=f32)
  pk = pools[...]
  m_tag = (tags[...] != 0).astype(f32)
  inv_t = 1.0 / (jnp.sum(m_tag, axis=1, keepdims=True) + 1e-08)
  iv += jnp.dot(pk[:, 0:16] * inv_t, Wi[96:112], preferred_element_type=f32)
  m_ts = (tastes[...] != 0).astype(f32)
  inv_s = 1.0 / (jnp.sum(m_ts, axis=1, keepdims=True) + 1e-08)
  iv += jnp.dot(pk[:, 16:32] * inv_s, Wi[112:128], preferred_element_type=f32)
  iv += jnp.dot(pk[:, 32:48], Wi[128:144], preferred_element_type=f32)
  iv += price[...] * jnp.dot(price_W[...], Wi[144:160], preferred_element_type=f32)
  iv += order[...] * jnp.dot(order_W[...], Wi[160:168], preferred_element_type=f32)
  iv += rating[...] * jnp.dot(rating_W[...], Wi[168:176], preferred_element_type=f32)
  iv += jnp.dot(iloc[...], jnp.dot(iloc_W[...], Wi[176:192],
                                   preferred_element_type=f32),
                preferred_element_type=f32)
  iv += itime[...] * jnp.dot(itime_W[...], Wi[192:200], preferred_element_type=f32)
  id1h = (iday[...] == lax.broadcasted_iota(jnp.int32, (_BC, 7), 1)).astype(f32)
  iv += jnp.dot(id1h, jnp.dot(iday_tab[...], Wi[200:208],
                              preferred_element_type=f32),
                preferred_element_type=f32)
  ibias = (jnp.dot(price_b[...], Wi[144:160], preferred_element_type=f32)
           + jnp.dot(order_b[...], Wi[160:168], preferred_element_type=f32)
           + jnp.dot(rating_b[...], Wi[168:176], preferred_element_type=f32)
           + jnp.dot(iloc_b[...], Wi[176:192], preferred_element_type=f32)
           + jnp.dot(itime_b[...], Wi[192:200], preferred_element_type=f32)
           + ip_b[...])
  iv += ibias

  un = uv / jnp.maximum(jnp.sqrt(jnp.sum(uv * uv, axis=-1, keepdims=True)), 1e-12)
  it = iv / jnp.maximum(jnp.sqrt(jnp.sum(iv * iv, axis=-1, keepdims=True)), 1e-12)
  un_out[...] = un
  it_out[...] = it
  sc_out[...] = jnp.sum(un * it, axis=-1, keepdims=True)


def _chunk(d):
  return pl.BlockSpec((_BC, d), lambda i: (i, 0))


def _full(shape):
  return pl.BlockSpec(shape, lambda i: (0,) * len(shape))


def kernel(user_id, age, gender, user_location, user_time_of_day,
           user_day_of_week, recency, dish_id, store_id, tags, tastes,
           category, price, order_times, rating, item_location,
           item_time_of_day, item_day_of_week, user_emb_table, user_age_W,
           user_age_b, user_gender_table, user_location_W, user_location_b,
           user_time_W, user_time_b, user_day_table, user_recency_W,
           user_recency_b, dish_emb_table, store_emb_table, tag_emb_table,
           taste_emb_table, cat_emb_table, dish_price_W, dish_price_b,
           dish_order_times_W, dish_order_times_b, dish_rating_W,
           dish_rating_b, dish_location_W, dish_location_b, dish_time_W,
           dish_time_b, dish_day_table, user_proj_W, user_proj_b,
           item_proj_W, item_proj_b):
  i32 = jnp.int32
  uid1 = user_id.astype(i32)
  did1 = dish_id.astype(i32)
  sid1 = store_id.astype(i32)
  tag1 = tags.astype(i32).reshape(_B * 5)
  tas1 = tastes.astype(i32).reshape(_B * 3)
  cat1 = category.astype(i32)

  u128 = user_emb_table.reshape(-1, 128)
  d128 = dish_emb_table.reshape(-1, 128)
  s128 = store_emb_table.reshape(-1, 128)
  t1 = tag_emb_table.reshape(-1)
  ts1 = taste_emb_table.reshape(-1)
  c1 = cat_emb_table.reshape(-1)

  urows, drows, srows, pools = _sc_gather_kernel()(
      u128, uid1, d128, did1, s128, sid1, t1, tag1, ts1, tas1, c1, cat1)

  grid = (_B // _BC,)
  un, it, sc = pl.pallas_call(
      _tc_body,
      grid=grid,
      in_specs=[
          _chunk(128), _chunk(128), _chunk(128), _chunk(128),
          _chunk(1), _chunk(1), _chunk(1),
          _chunk(1), _chunk(1), _chunk(2), _chunk(1), _chunk(1), _chunk(1),
          _chunk(5), _chunk(3), _chunk(1), _chunk(1), _chunk(1), _chunk(2),
          _chunk(1), _chunk(1),
          _full((1, 16)), _full((1, 16)), _full((3, 16)), _full((2, 16)),
          _full((1, 16)), _full((1, 8)), _full((1, 8)), _full((7, 8)),
          _full((1, 16)), _full((1, 16)),
          _full((1, 16)), _full((1, 16)), _full((1, 8)), _full((1, 8)),
          _full((1, 8)), _full((1, 8)), _full((2, 16)), _full((1, 16)),
          _full((1, 8)), _full((1, 8)), _full((7, 8)),
          _full((144, 64)), _full((1, 64)), _full((208, 64)), _full((1, 64)),
      ],
      out_specs=[_chunk(64), _chunk(64), _chunk(1)],
      out_shape=[
          jax.ShapeDtypeStruct((_B, 64), jnp.float32),
          jax.ShapeDtypeStruct((_B, 64), jnp.float32),
          jax.ShapeDtypeStruct((_B, 1), jnp.float32),
      ],
  )(
      urows, drows, srows, pools,
      uid1.reshape(_B, 1), did1.reshape(_B, 1), sid1.reshape(_B, 1),
      age.reshape(_B, 1), gender.astype(i32).reshape(_B, 1), user_location,
      user_time_of_day.reshape(_B, 1),
      user_day_of_week.astype(i32).reshape(_B, 1), recency.reshape(_B, 1),
      tags.astype(i32), tastes.astype(i32), price.reshape(_B, 1),
      order_times.reshape(_B, 1), rating.reshape(_B, 1), item_location,
      item_time_of_day.reshape(_B, 1),
      item_day_of_week.astype(i32).reshape(_B, 1),
      user_age_W, user_age_b.reshape(1, 16), user_gender_table,
      user_location_W, user_location_b.reshape(1, 16), user_time_W,
      user_time_b.reshape(1, 8), user_day_table, user_recency_W,
      user_recency_b.reshape(1, 16),
      dish_price_W, dish_price_b.reshape(1, 16), dish_order_times_W,
      dish_order_times_b.reshape(1, 8), dish_rating_W,
      dish_rating_b.reshape(1, 8), dish_location_W,
      dish_location_b.reshape(1, 16), dish_time_W, dish_time_b.reshape(1, 8),
      dish_day_table,
      user_proj_W, user_proj_b.reshape(1, 64), item_proj_W,
      item_proj_b.reshape(1, 64),
  )
  return (un, it, sc.reshape(_B))


# speedup vs baseline: 1.0245x; 1.0054x over previous
import functools

import jax
import jax.numpy as jnp
from jax import lax
from jax.experimental import pallas as pl
from jax.experimental.pallas import tpu as pltpu
from jax.experimental.pallas import tpu_sc as plsc

_B = 16384
_NC = 2
_NS = 16
_NW = _NC * _NS
_PW = _B // _NW
_H = 4
_QB = _PW // _H

_BC = 1024


def _pool_chunk(idbuf, tabv, sumv, q, slots):
  i32 = jnp.int32
  f32 = jnp.float32
  iota = lax.iota(i32, 16)

  def body(g, carry):
    sb = q * _QB + g * 16
    accs = [jnp.zeros((16,), f32) for _ in range(16)]
    for j in range(slots):
      pvec = iota * slots + (sb * slots + j)
      tv = plsc.load_gather(idbuf, [pvec])
      mf = (tv != 0).astype(f32)
      fbase = tv * 16
      for d in range(16):
        vals = plsc.load_gather(tabv, [fbase + d])
        accs[d] = accs[d] + vals * mf
    rowv = g * 16 + iota
    for d in range(16):
      plsc.store_scatter(sumv, [rowv, jnp.full((16,), d, i32)], accs[d])
    return carry

  lax.fori_loop(0, _QB // 16, body, 0)


def _lookup_chunk(idbuf, tabv, sumv, q):
  i32 = jnp.int32
  iota = lax.iota(i32, 16)

  def body(g, carry):
    sb = q * _QB + g * 16
    cv = idbuf[pl.ds(sb, 16)]
    fbase = cv * 16
    rowv = g * 16 + iota
    for d in range(16):
      vals = plsc.load_gather(tabv, [fbase + d])
      plsc.store_scatter(sumv, [rowv, jnp.full((16,), d, i32)], vals)
    return carry

  lax.fori_loop(0, _QB // 16, body, 0)


def _sc_body(utab, uid, dtab, did, stab, sid, ttab, tagid, tstab, tasteid,
             ctab, catid,
             urows, drows, srows, tagsums, tastesums, catrows,
             tagtab_v, tastetab_v, cattab_v, it1, its1, ict1,
             iu, idh, ist, ru, rd, rs, tsum, tssum, csum,
             sT, sI, sG, sO):
  wid = lax.axis_index("s") * _NC + lax.axis_index("c")
  b0 = wid * _PW
  pre = [
      pltpu.async_copy(ttab, tagtab_v, sT),
      pltpu.async_copy(tstab, tastetab_v, sT),
      pltpu.async_copy(ctab, cattab_v, sT),
      pltpu.async_copy(tagid.at[pl.ds(b0 * 5, _PW * 5)], it1, sT),
      pltpu.async_copy(tasteid.at[pl.ds(b0 * 3, _PW * 3)], its1, sT),
      pltpu.async_copy(catid.at[pl.ds(b0, _PW)], ict1, sT),
  ]
  for cp in pre:
    cp.wait()
  for q in range(_H):
    b = b0 + q * _QB
    ics = [
        pltpu.async_copy(uid.at[pl.ds(b, _QB)], iu.at[0], sI),
        pltpu.async_copy(did.at[pl.ds(b, _QB)], idh.at[0], sI),
        pltpu.async_copy(sid.at[pl.ds(b, _QB)], ist.at[0], sI),
    ]
    for cp in ics:
      cp.wait()
    gcs = [
        pltpu.async_copy(utab.at[iu.at[0]], ru, sG),
        pltpu.async_copy(dtab.at[idh.at[0]], rd, sG),
        pltpu.async_copy(stab.at[ist.at[0]], rs, sG),
    ]
    _pool_chunk(it1, tagtab_v, tsum, q, 5)
    _pool_chunk(its1, tastetab_v, tssum, q, 3)
    _lookup_chunk(ict1, cattab_v, csum, q)
    for cp in gcs:
      cp.wait()
    ocs = [
        pltpu.async_copy(ru, urows.at[pl.ds(b, _QB)], sO),
        pltpu.async_copy(rd, drows.at[pl.ds(b, _QB)], sO),
        pltpu.async_copy(rs, srows.at[pl.ds(b, _QB)], sO),
        pltpu.async_copy(tsum, tagsums.at[pl.ds(b, _QB)], sO),
        pltpu.async_copy(tssum, tastesums.at[pl.ds(b, _QB)], sO),
        pltpu.async_copy(csum, catrows.at[pl.ds(b, _QB)], sO),
    ]
    for cp in ocs:
      cp.wait()


@functools.cache
def _sc_gather_kernel():
  f32 = jnp.float32
  i32 = jnp.int32
  return pl.kernel(
      _sc_body,
      out_type=[
          jax.ShapeDtypeStruct((_B, 64), f32),
          jax.ShapeDtypeStruct((_B, 64), f32),
          jax.ShapeDtypeStruct((_B, 32), f32),
          jax.ShapeDtypeStruct((_B, 16), f32),
          jax.ShapeDtypeStruct((_B, 16), f32),
          jax.ShapeDtypeStruct((_B, 16), f32),
      ],
      mesh=plsc.VectorSubcoreMesh(core_axis_name="c", subcore_axis_name="s",
                                  num_cores=_NC, num_subcores=_NS),
      scratch_types=[
          pltpu.VMEM((16000,), f32),
          pltpu.VMEM((16000,), f32),
          pltpu.VMEM((16000,), f32),
          pltpu.VMEM((_PW * 5,), i32),
          pltpu.VMEM((_PW * 3,), i32),
          pltpu.VMEM((_PW,), i32),
          pltpu.VMEM((1, _QB), i32),
          pltpu.VMEM((1, _QB), i32),
          pltpu.VMEM((1, _QB), i32),
          pltpu.VMEM((_QB, 64), f32),
          pltpu.VMEM((_QB, 64), f32),
          pltpu.VMEM((_QB, 32), f32),
          pltpu.VMEM((_QB, 16), f32),
          pltpu.VMEM((_QB, 16), f32),
          pltpu.VMEM((_QB, 16), f32),
          pltpu.SemaphoreType.DMA,
          pltpu.SemaphoreType.DMA,
          pltpu.SemaphoreType.DMA,
          pltpu.SemaphoreType.DMA,
      ],
      compiler_params=pltpu.CompilerParams(use_tc_tiling_on_sc=False,
                                           needs_layout_passes=False),
  )


def _tc_body(urows, drows, srows, tagsums, tastesums, catrows,
             age, gender, uloc, utime, uday, rec,
             tags, tastes, price, order, rating, iloc, itime, iday,
             age_W, age_b, gender_tab, uloc_W, uloc_b, utime_W, utime_b,
             uday_tab, rec_W, rec_b,
             price_W, price_b, order_W, order_b, rating_W, rating_b,
             iloc_W, iloc_b, itime_W, itime_b, iday_tab,
             up_W, up_b, ip_W, ip_b,
             un_out, it_out, sc_out):
  f32 = jnp.float32
  Wu = up_W[...]
  Wi = ip_W[...]

  uv = jnp.dot(urows[...], Wu[0:64], preferred_element_type=f32)
  uv += age[...] * jnp.dot(age_W[...], Wu[64:80], preferred_element_type=f32)
  g1h = (gender[...] == lax.broadcasted_iota(jnp.int32, (_BC, 3), 1)).astype(f32)
  uv += jnp.dot(g1h, jnp.dot(gender_tab[...], Wu[80:96],
                             preferred_element_type=f32),
                preferred_element_type=f32)
  uv += jnp.dot(uloc[...], jnp.dot(uloc_W[...], Wu[96:112],
                                   preferred_element_type=f32),
                preferred_element_type=f32)
  uv += utime[...] * jnp.dot(utime_W[...], Wu[112:120], preferred_element_type=f32)
  ud1h = (uday[...] == lax.broadcasted_iota(jnp.int32, (_BC, 7), 1)).astype(f32)
  uv += jnp.dot(ud1h, jnp.dot(uday_tab[...], Wu[120:128],
                              preferred_element_type=f32),
                preferred_element_type=f32)
  uv += rec[...] * jnp.dot(rec_W[...], Wu[128:144], preferred_element_type=f32)
  ubias = (jnp.dot(age_b[...], Wu[64:80], preferred_element_type=f32)
           + jnp.dot(uloc_b[...], Wu[96:112], preferred_element_type=f32)
           + jnp.dot(utime_b[...], Wu[112:120], preferred_element_type=f32)
           + jnp.dot(rec_b[...], Wu[128:144], preferred_element_type=f32)
           + up_b[...])
  uv += ubias

  iv = jnp.dot(drows[...], Wi[0:64], preferred_element_type=f32)
  iv += jnp.dot(srows[...], Wi[64:96], preferred_element_type=f32)
  m_tag = (tags[...] != 0).astype(f32)
  inv_t = 1.0 / (jnp.sum(m_tag, axis=1, keepdims=True) + 1e-08)
  iv += jnp.dot(tagsums[...] * inv_t, Wi[96:112], preferred_element_type=f32)
  m_ts = (tastes[...] != 0).astype(f32)
  inv_s = 1.0 / (jnp.sum(m_ts, axis=1, keepdims=True) + 1e-08)
  iv += jnp.dot(tastesums[...] * inv_s, Wi[112:128], preferred_element_type=f32)
  iv += jnp.dot(catrows[...], Wi[128:144], preferred_element_type=f32)
  iv += price[...] * jnp.dot(price_W[...], Wi[144:160], preferred_element_type=f32)
  iv += order[...] * jnp.dot(order_W[...], Wi[160:168], preferred_element_type=f32)
  iv += rating[...] * jnp.dot(rating_W[...], Wi[168:176], preferred_element_type=f32)
  iv += jnp.dot(iloc[...], jnp.dot(iloc_W[...], Wi[176:192],
                                   preferred_element_type=f32),
                preferred_element_type=f32)
  iv += itime[...] * jnp.dot(itime_W[...], Wi[192:200], preferred_element_type=f32)
  id1h = (iday[...] == lax.broadcasted_iota(jnp.int32, (_BC, 7), 1)).astype(f32)
  iv += jnp.dot(id1h, jnp.dot(iday_tab[...], Wi[200:208],
                              preferred_element_type=f32),
                preferred_element_type=f32)
  ibias = (jnp.dot(price_b[...], Wi[144:160], preferred_element_type=f32)
           + jnp.dot(order_b[...], Wi[160:168], preferred_element_type=f32)
           + jnp.dot(rating_b[...], Wi[168:176], preferred_element_type=f32)
           + jnp.dot(iloc_b[...], Wi[176:192], preferred_element_type=f32)
           + jnp.dot(itime_b[...], Wi[192:200], preferred_element_type=f32)
           + ip_b[...])
  iv += ibias

  un = uv / jnp.maximum(jnp.sqrt(jnp.sum(uv * uv, axis=-1, keepdims=True)), 1e-12)
  it = iv / jnp.maximum(jnp.sqrt(jnp.sum(iv * iv, axis=-1, keepdims=True)), 1e-12)
  un_out[...] = un
  it_out[...] = it
  sc_out[...] = jnp.sum(un * it, axis=-1, keepdims=True)


def _chunk(d):
  return pl.BlockSpec((_BC, d), lambda i: (i, 0))


def _full(shape):
  return pl.BlockSpec(shape, lambda i: (0,) * len(shape))


def kernel(user_id, age, gender, user_location, user_time_of_day,
           user_day_of_week, recency, dish_id, store_id, tags, tastes,
           category, price, order_times, rating, item_location,
           item_time_of_day, item_day_of_week, user_emb_table, user_age_W,
           user_age_b, user_gender_table, user_location_W, user_location_b,
           user_time_W, user_time_b, user_day_table, user_recency_W,
           user_recency_b, dish_emb_table, store_emb_table, tag_emb_table,
           taste_emb_table, cat_emb_table, dish_price_W, dish_price_b,
           dish_order_times_W, dish_order_times_b, dish_rating_W,
           dish_rating_b, dish_location_W, dish_location_b, dish_time_W,
           dish_time_b, dish_day_table, user_proj_W, user_proj_b,
           item_proj_W, item_proj_b):
  i32 = jnp.int32
  uid1 = user_id.astype(i32)
  did1 = dish_id.astype(i32)
  sid1 = store_id.astype(i32)
  tag1 = tags.astype(i32).reshape(_B * 5)
  tas1 = tastes.astype(i32).reshape(_B * 3)
  cat1 = category.astype(i32)

  t1 = tag_emb_table.reshape(-1)
  ts1 = taste_emb_table.reshape(-1)
  c1 = cat_emb_table.reshape(-1)

  urows, drows, srows, tagsums, tastesums, catrows = _sc_gather_kernel()(
      user_emb_table, uid1, dish_emb_table, did1, store_emb_table, sid1,
      t1, tag1, ts1, tas1, c1, cat1)

  grid = (_B // _BC,)
  un, it, sc = pl.pallas_call(
      _tc_body,
      grid=grid,
      in_specs=[
          _chunk(64), _chunk(64), _chunk(32), _chunk(16), _chunk(16),
          _chunk(16),
          _chunk(1), _chunk(1), _chunk(2), _chunk(1), _chunk(1), _chunk(1),
          _chunk(5), _chunk(3), _chunk(1), _chunk(1), _chunk(1), _chunk(2),
          _chunk(1), _chunk(1),
          _full((1, 16)), _full((1, 16)), _full((3, 16)), _full((2, 16)),
          _full((1, 16)), _full((1, 8)), _full((1, 8)), _full((7, 8)),
          _full((1, 16)), _full((1, 16)),
          _full((1, 16)), _full((1, 16)), _full((1, 8)), _full((1, 8)),
          _full((1, 8)), _full((1, 8)), _full((2, 16)), _full((1, 16)),
          _full((1, 8)), _full((1, 8)), _full((7, 8)),
          _full((144, 64)), _full((1, 64)), _full((208, 64)), _full((1, 64)),
      ],
      out_specs=[_chunk(64), _chunk(64), _chunk(1)],
      out_shape=[
          jax.ShapeDtypeStruct((_B, 64), jnp.float32),
          jax.ShapeDtypeStruct((_B, 64), jnp.float32),
          jax.ShapeDtypeStruct((_B, 1), jnp.float32),
      ],
  )(
      urows, drows, srows, tagsums, tastesums, catrows,
      age.reshape(_B, 1), gender.astype(i32).reshape(_B, 1), user_location,
      user_time_of_day.reshape(_B, 1),
      user_day_of_week.astype(i32).reshape(_B, 1), recency.reshape(_B, 1),
      tags.astype(i32), tastes.astype(i32), price.reshape(_B, 1),
      order_times.reshape(_B, 1), rating.reshape(_B, 1), item_location,
      item_time_of_day.reshape(_B, 1),
      item_day_of_week.astype(i32).reshape(_B, 1),
      user_age_W, user_age_b.reshape(1, 16), user_gender_table,
      user_location_W, user_location_b.reshape(1, 16), user_time_W,
      user_time_b.reshape(1, 8), user_day_table, user_recency_W,
      user_recency_b.reshape(1, 16),
      dish_price_W, dish_price_b.reshape(1, 16), dish_order_times_W,
      dish_order_times_b.reshape(1, 8), dish_rating_W,
      dish_rating_b.reshape(1, 8), dish_location_W,
      dish_location_b.reshape(1, 16), dish_time_W, dish_time_b.reshape(1, 8),
      dish_day_table,
      user_proj_W, user_proj_b.reshape(1, 64), item_proj_W,
      item_proj_b.reshape(1, 64),
  )
  return (un, it, sc.reshape(_B))


# speedup vs baseline: 1.2867x; 1.2559x over previous
import functools

import jax
import jax.numpy as jnp
from jax import lax
from jax.experimental import pallas as pl
from jax.experimental.pallas import tpu as pltpu
from jax.experimental.pallas import tpu_sc as plsc

_B = 16384
_NC = 2
_NS = 16
_NW = _NC * _NS
_PW = _B // _NW
_QB = 16
_NQ = _PW // _QB

_BC = 1024


def _pool_group(idbuf, tabv, dstv, sb, slots, dofs):
  i32 = jnp.int32
  f32 = jnp.float32
  iota = lax.iota(i32, 16)
  accs = [jnp.zeros((16,), f32) for _ in range(16)]
  for j in range(slots):
    pvec = iota * slots + (sb * slots + j)
    tv = plsc.load_gather(idbuf, [pvec])
    mf = (tv != 0).astype(f32)
    fbase = tv * 16
    for d in range(16):
      vals = plsc.load_gather(tabv, [fbase + d])
      accs[d] = accs[d] + vals * mf
  for d in range(16):
    plsc.store_scatter(dstv, [iota, jnp.full((16,), dofs + d, i32)], accs[d])


def _lookup_group(idbuf, tabv, dstv, sb, dofs):
  i32 = jnp.int32
  iota = lax.iota(i32, 16)
  cv = idbuf[pl.ds(sb, 16)]
  fbase = cv * 16
  for d in range(16):
    vals = plsc.load_gather(tabv, [fbase + d])
    plsc.store_scatter(dstv, [iota, jnp.full((16,), dofs + d, i32)], vals)


def _sc_body(utab, uid, dtab, did, stab, sid, ttab, tagid, tstab, tasteid,
             ctab, catid,
             out1, out2,
             tagtab_v, tastetab_v, cattab_v, it1, its1, ict1,
             iv_v, tbuf, dbuf, sbuf, cb1, cb2,
             sT, sI, sG, sO):
  i32 = jnp.int32
  wid = lax.axis_index("s") * _NC + lax.axis_index("c")
  b0 = wid * _PW
  pre = [
      pltpu.async_copy(ttab, tagtab_v, sT),
      pltpu.async_copy(tstab, tastetab_v, sT),
      pltpu.async_copy(ctab, cattab_v, sT),
      pltpu.async_copy(tagid.at[pl.ds(b0 * 5, _PW * 5)], it1, sT),
      pltpu.async_copy(tasteid.at[pl.ds(b0 * 3, _PW * 3)], its1, sT),
      pltpu.async_copy(catid.at[pl.ds(b0, _PW)], ict1, sT),
  ]
  for cp in pre:
    cp.wait()

  def chunk(q, carry):
    b = b0 + q * _QB
    sb = q * _QB
    ics = [
        pltpu.async_copy(uid.at[pl.ds(b, _QB)], iv_v.at[0], sI),
        pltpu.async_copy(did.at[pl.ds(b, _QB)], iv_v.at[1], sI),
        pltpu.async_copy(sid.at[pl.ds(b, _QB)], iv_v.at[2], sI),
    ]
    for cp in ics:
      cp.wait()
    uvec = iv_v[0, pl.ds(0, 16)]
    dvec = iv_v[1, pl.ds(0, 16)]
    svec = iv_v[2, pl.ds(0, 16)]
    ubase = lax.shift_right_logical(uvec, 3) * 8
    dbase = lax.shift_right_logical(dvec, 3) * 8
    sbase = lax.shift_right_logical(svec, 3) * 8
    gcs = []
    for s in range(_QB):
      ub = pl.multiple_of(ubase[s], 8)
      gcs.append(pltpu.async_copy(utab.at[pl.ds(ub, 8)], tbuf.at[s], sG))
    for s in range(_QB):
      db = pl.multiple_of(dbase[s], 8)
      gcs.append(pltpu.async_copy(dtab.at[pl.ds(db, 8)], dbuf.at[s], sG))
    for s in range(_QB):
      sb2 = pl.multiple_of(sbase[s], 8)
      gcs.append(pltpu.async_copy(stab.at[pl.ds(sb2, 8)], sbuf.at[s], sG))
    _pool_group(it1, tagtab_v, cb2, sb, 5, 32)
    _pool_group(its1, tastetab_v, cb2, sb, 3, 48)
    _lookup_group(ict1, cattab_v, cb2, sb, 64)
    for cp in gcs:
      cp.wait()
    ursel = lax.bitwise_and(uvec, 7)
    drsel = lax.bitwise_and(dvec, 7)
    srsel = lax.bitwise_and(svec, 7)
    for s in range(_QB):
      ur = ursel[s]
      dr = drsel[s]
      sr = srsel[s]
      for k in range(4):
        sl = pl.ds(k * 16, 16)
        cb1[s, sl] = tbuf[s, ur, sl]
      for k in range(4):
        sl = pl.ds(k * 16, 16)
        cb1[s, pl.ds(64 + k * 16, 16)] = dbuf[s, dr, sl]
      for k in range(2):
        sl = pl.ds(k * 16, 16)
        cb2[s, sl] = sbuf[s, sr, sl]
    o1 = pltpu.async_copy(cb1, out1.at[pl.ds(b, _QB)], sO)
    o2 = pltpu.async_copy(cb2, out2.at[pl.ds(b, _QB)], sO)
    o1.wait()
    o2.wait()
    return carry

  lax.fori_loop(0, _NQ, chunk, 0)


@functools.cache
def _sc_gather_kernel():
  f32 = jnp.float32
  i32 = jnp.int32
  return pl.kernel(
      _sc_body,
      out_type=[
          jax.ShapeDtypeStruct((_B, 128), f32),
          jax.ShapeDtypeStruct((_B, 128), f32),
      ],
      mesh=plsc.VectorSubcoreMesh(core_axis_name="c", subcore_axis_name="s",
                                  num_cores=_NC, num_subcores=_NS),
      scratch_types=[
          pltpu.VMEM((16000,), f32),
          pltpu.VMEM((16000,), f32),
          pltpu.VMEM((16000,), f32),
          pltpu.VMEM((_PW * 5,), i32),
          pltpu.VMEM((_PW * 3,), i32),
          pltpu.VMEM((_PW,), i32),
          pltpu.VMEM((3, _QB), i32),
          pltpu.VMEM((_QB, 8, 64), f32),
          pltpu.VMEM((_QB, 8, 64), f32),
          pltpu.VMEM((_QB, 8, 32), f32),
          pltpu.VMEM((_QB, 128), f32),
          pltpu.VMEM((_QB, 128), f32),
          pltpu.SemaphoreType.DMA,
          pltpu.SemaphoreType.DMA,
          pltpu.SemaphoreType.DMA,
          pltpu.SemaphoreType.DMA,
      ],
      compiler_params=pltpu.CompilerParams(use_tc_tiling_on_sc=True,
                                           needs_layout_passes=False),
  )


def _tc_body(ud, sp,
             age, gender, uloc, utime, uday, rec,
             tags, tastes, price, order, rating, iloc, itime, iday,
             age_W, age_b, gender_tab, uloc_W, uloc_b, utime_W, utime_b,
             uday_tab, rec_W, rec_b,
             price_W, price_b, order_W, order_b, rating_W, rating_b,
             iloc_W, iloc_b, itime_W, itime_b, iday_tab,
             up_W, up_b, ip_W, ip_b,
             un_out, it_out, sc_out):
  f32 = jnp.float32
  Wu = up_W[...]
  Wi = ip_W[...]
  udv = ud[...]
  spv = sp[...]

  uv = jnp.dot(udv[:, 0:64], Wu[0:64], preferred_element_type=f32)
  uv += age[...] * jnp.dot(age_W[...], Wu[64:80], preferred_element_type=f32)
  g1h = (gender[...] == lax.broadcasted_iota(jnp.int32, (_BC, 3), 1)).astype(f32)
  uv += jnp.dot(g1h, jnp.dot(gender_tab[...], Wu[80:96],
                             preferred_element_type=f32),
                preferred_element_type=f32)
  uv += jnp.dot(uloc[...], jnp.dot(uloc_W[...], Wu[96:112],
                                   preferred_element_type=f32),
                preferred_element_type=f32)
  uv += utime[...] * jnp.dot(utime_W[...], Wu[112:120], preferred_element_type=f32)
  ud1h = (uday[...] == lax.broadcasted_iota(jnp.int32, (_BC, 7), 1)).astype(f32)
  uv += jnp.dot(ud1h, jnp.dot(uday_tab[...], Wu[120:128],
                              preferred_element_type=f32),
                preferred_element_type=f32)
  uv += rec[...] * jnp.dot(rec_W[...], Wu[128:144], preferred_element_type=f32)
  ubias = (jnp.dot(age_b[...], Wu[64:80], preferred_element_type=f32)
           + jnp.dot(uloc_b[...], Wu[96:112], preferred_element_type=f32)
           + jnp.dot(utime_b[...], Wu[112:120], preferred_element_type=f32)
           + jnp.dot(rec_b[...], Wu[128:144], preferred_element_type=f32)
           + up_b[...])
  uv += ubias

  iv = jnp.dot(udv[:, 64:128], Wi[0:64], preferred_element_type=f32)
  iv += jnp.dot(spv[:, 0:32], Wi[64:96], preferred_element_type=f32)
  m_tag = (tags[...] != 0).astype(f32)
  inv_t = 1.0 / (jnp.sum(m_tag, axis=1, keepdims=True) + 1e-08)
  iv += jnp.dot(spv[:, 32:48] * inv_t, Wi[96:112], preferred_element_type=f32)
  m_ts = (tastes[...] != 0).astype(f32)
  inv_s = 1.0 / (jnp.sum(m_ts, axis=1, keepdims=True) + 1e-08)
  iv += jnp.dot(spv[:, 48:64] * inv_s, Wi[112:128], preferred_element_type=f32)
  iv += jnp.dot(spv[:, 64:80], Wi[128:144], preferred_element_type=f32)
  iv += price[...] * jnp.dot(price_W[...], Wi[144:160], preferred_element_type=f32)
  iv += order[...] * jnp.dot(order_W[...], Wi[160:168], preferred_element_type=f32)
  iv += rating[...] * jnp.dot(rating_W[...], Wi[168:176], preferred_element_type=f32)
  iv += jnp.dot(iloc[...], jnp.dot(iloc_W[...], Wi[176:192],
                                   preferred_element_type=f32),
                preferred_element_type=f32)
  iv += itime[...] * jnp.dot(itime_W[...], Wi[192:200], preferred_element_type=f32)
  id1h = (iday[...] == lax.broadcasted_iota(jnp.int32, (_BC, 7), 1)).astype(f32)
  iv += jnp.dot(id1h, jnp.dot(iday_tab[...], Wi[200:208],
                              preferred_element_type=f32),
                preferred_element_type=f32)
  ibias = (jnp.dot(price_b[...], Wi[144:160], preferred_element_type=f32)
           + jnp.dot(order_b[...], Wi[160:168], preferred_element_type=f32)
           + jnp.dot(rating_b[...], Wi[168:176], preferred_element_type=f32)
           + jnp.dot(iloc_b[...], Wi[176:192], preferred_element_type=f32)
           + jnp.dot(itime_b[...], Wi[192:200], preferred_element_type=f32)
           + ip_b[...])
  iv += ibias

  un = uv / jnp.maximum(jnp.sqrt(jnp.sum(uv * uv, axis=-1, keepdims=True)), 1e-12)
  it = iv / jnp.maximum(jnp.sqrt(jnp.sum(iv * iv, axis=-1, keepdims=True)), 1e-12)
  un_out[...] = un
  it_out[...] = it
  sc_out[...] = jnp.sum(un * it, axis=-1, keepdims=True)


def _chunk(d):
  return pl.BlockSpec((_BC, d), lambda i: (i, 0))


def _full(shape):
  return pl.BlockSpec(shape, lambda i: (0,) * len(shape))


def kernel(user_id, age, gender, user_location, user_time_of_day,
           user_day_of_week, recency, dish_id, store_id, tags, tastes,
           category, price, order_times, rating, item_location,
           item_time_of_day, item_day_of_week, user_emb_table, user_age_W,
           user_age_b, user_gender_table, user_location_W, user_location_b,
           user_time_W, user_time_b, user_day_table, user_recency_W,
           user_recency_b, dish_emb_table, store_emb_table, tag_emb_table,
           taste_emb_table, cat_emb_table, dish_price_W, dish_price_b,
           dish_order_times_W, dish_order_times_b, dish_rating_W,
           dish_rating_b, dish_location_W, dish_location_b, dish_time_W,
           dish_time_b, dish_day_table, user_proj_W, user_proj_b,
           item_proj_W, item_proj_b):
  i32 = jnp.int32
  uid1 = user_id.astype(i32)
  did1 = dish_id.astype(i32)
  sid1 = store_id.astype(i32)
  tag1 = tags.astype(i32).reshape(_B * 5)
  tas1 = tastes.astype(i32).reshape(_B * 3)
  cat1 = category.astype(i32)

  t1 = tag_emb_table.reshape(-1)
  ts1 = taste_emb_table.reshape(-1)
  c1 = cat_emb_table.reshape(-1)

  ud, sp = _sc_gather_kernel()(
      user_emb_table, uid1, dish_emb_table, did1, store_emb_table, sid1,
      t1, tag1, ts1, tas1, c1, cat1)

  grid = (_B // _BC,)
  un, it, sc = pl.pallas_call(
      _tc_body,
      grid=grid,
      in_specs=[
          _chunk(128), _chunk(128),
          _chunk(1), _chunk(1), _chunk(2), _chunk(1), _chunk(1), _chunk(1),
          _chunk(5), _chunk(3), _chunk(1), _chunk(1), _chunk(1), _chunk(2),
          _chunk(1), _chunk(1),
          _full((1, 16)), _full((1, 16)), _full((3, 16)), _full((2, 16)),
          _full((1, 16)), _full((1, 8)), _full((1, 8)), _full((7, 8)),
          _full((1, 16)), _full((1, 16)),
          _full((1, 16)), _full((1, 16)), _full((1, 8)), _full((1, 8)),
          _full((1, 8)), _full((1, 8)), _full((2, 16)), _full((1, 16)),
          _full((1, 8)), _full((1, 8)), _full((7, 8)),
          _full((144, 64)), _full((1, 64)), _full((208, 64)), _full((1, 64)),
      ],
      out_specs=[_chunk(64), _chunk(64), _chunk(1)],
      out_shape=[
          jax.ShapeDtypeStruct((_B, 64), jnp.float32),
          jax.ShapeDtypeStruct((_B, 64), jnp.float32),
          jax.ShapeDtypeStruct((_B, 1), jnp.float32),
      ],
  )(
      ud, sp,
      age.reshape(_B, 1), gender.astype(i32).reshape(_B, 1), user_location,
      user_time_of_day.reshape(_B, 1),
      user_day_of_week.astype(i32).reshape(_B, 1), recency.reshape(_B, 1),
      tags.astype(i32), tastes.astype(i32), price.reshape(_B, 1),
      order_times.reshape(_B, 1), rating.reshape(_B, 1), item_location,
      item_time_of_day.reshape(_B, 1),
      item_day_of_week.astype(i32).reshape(_B, 1),
      user_age_W, user_age_b.reshape(1, 16), user_gender_table,
      user_location_W, user_location_b.reshape(1, 16), user_time_W,
      user_time_b.reshape(1, 8), user_day_table, user_recency_W,
      user_recency_b.reshape(1, 16),
      dish_price_W, dish_price_b.reshape(1, 16), dish_order_times_W,
      dish_order_times_b.reshape(1, 8), dish_rating_W,
      dish_rating_b.reshape(1, 8), dish_location_W,
      dish_location_b.reshape(1, 16), dish_time_W, dish_time_b.reshape(1, 8),
      dish_day_table,
      user_proj_W, user_proj_b.reshape(1, 64), item_proj_W,
      item_proj_b.reshape(1, 64),
  )
  return (un, it, sc.reshape(_B))


# speedup vs baseline: 1.3370x; 1.0391x over previous
import functools

import jax
import jax.numpy as jnp
from jax import lax
from jax.experimental import pallas as pl
from jax.experimental.pallas import tpu as pltpu
from jax.experimental.pallas import tpu_sc as plsc

_B = 16384
_NC = 2
_NS = 16
_NW = _NC * _NS
_PW = _B // _NW
_QB = 32
_NQ = _PW // _QB

_BC = 1024


def _pool_group(idbuf, tabv, dstv, sb, slots, dofs):
  i32 = jnp.int32
  f32 = jnp.float32
  iota = lax.iota(i32, 16)
  accs = [jnp.zeros((16,), f32) for _ in range(16)]
  for j in range(slots):
    pvec = iota * slots + (sb * slots + j)
    tv = plsc.load_gather(idbuf, [pvec])
    mf = (tv != 0).astype(f32)
    fbase = tv * 16
    for d in range(16):
      vals = plsc.load_gather(tabv, [fbase + d])
      accs[d] = accs[d] + vals * mf
  rowv = sb + iota
  for d in range(16):
    plsc.store_scatter(dstv, [rowv, jnp.full((16,), dofs + d, i32)], accs[d])


def _lookup_group(idbuf, tabv, dstv, sb, dofs):
  i32 = jnp.int32
  iota = lax.iota(i32, 16)
  cv = idbuf[pl.ds(sb, 16)]
  fbase = cv * 16
  rowv = sb + iota
  for d in range(16):
    vals = plsc.load_gather(tabv, [fbase + d])
    plsc.store_scatter(dstv, [rowv, jnp.full((16,), dofs + d, i32)], vals)


def _sc_pool_body(ttab, tagid, tstab, tasteid, ctab, catid,
                  pools,
                  tagtab_v, tastetab_v, cattab_v, it1, its1, ict1, psum,
                  sT, sO):
  wid = lax.axis_index("s") * _NC + lax.axis_index("c")
  b0 = wid * _PW
  pre = [
      pltpu.async_copy(ttab, tagtab_v, sT),
      pltpu.async_copy(tstab, tastetab_v, sT),
      pltpu.async_copy(ctab, cattab_v, sT),
      pltpu.async_copy(tagid.at[pl.ds(b0 * 5, _PW * 5)], it1, sT),
      pltpu.async_copy(tasteid.at[pl.ds(b0 * 3, _PW * 3)], its1, sT),
      pltpu.async_copy(catid.at[pl.ds(b0, _PW)], ict1, sT),
  ]
  for cp in pre:
    cp.wait()

  def grp(g, carry):
    sb = g * 16
    _pool_group(it1, tagtab_v, psum, sb, 5, 0)
    _pool_group(its1, tastetab_v, psum, sb, 3, 16)
    _lookup_group(ict1, cattab_v, psum, sb, 32)
    return carry

  lax.fori_loop(0, _PW // 16, grp, 0)
  pltpu.async_copy(psum, pools.at[pl.ds(b0, _PW)], sO).wait()


@functools.cache
def _sc_pool_kernel():
  f32 = jnp.float32
  i32 = jnp.int32
  return pl.kernel(
      _sc_pool_body,
      out_type=[jax.ShapeDtypeStruct((_B, 48), f32)],
      mesh=plsc.VectorSubcoreMesh(core_axis_name="c", subcore_axis_name="s",
                                  num_cores=_NC, num_subcores=_NS),
      scratch_types=[
          pltpu.VMEM((16000,), f32),
          pltpu.VMEM((16000,), f32),
          pltpu.VMEM((16000,), f32),
          pltpu.VMEM((_PW * 5,), i32),
          pltpu.VMEM((_PW * 3,), i32),
          pltpu.VMEM((_PW,), i32),
          pltpu.VMEM((_PW, 48), f32),
          pltpu.SemaphoreType.DMA,
          pltpu.SemaphoreType.DMA,
      ],
      compiler_params=pltpu.CompilerParams(use_tc_tiling_on_sc=False,
                                           needs_layout_passes=False),
  )


def _sc_body(utab, uid, dtab, did, stab, sid,
             out1, out2,
             iv_v, tbuf, dbuf, sbuf, cb1, cb2,
             sI, sG, sO):
  i32 = jnp.int32
  wid = lax.axis_index("s") * _NC + lax.axis_index("c")
  b0 = wid * _PW

  def chunk(q, carry):
    b = b0 + q * _QB
    ics = [
        pltpu.async_copy(uid.at[pl.ds(b, _QB)], iv_v.at[0], sI),
        pltpu.async_copy(did.at[pl.ds(b, _QB)], iv_v.at[1], sI),
        pltpu.async_copy(sid.at[pl.ds(b, _QB)], iv_v.at[2], sI),
    ]
    for cp in ics:
      cp.wait()
    gcs = []
    uvecs, dvecs, svecs = [], [], []
    for h in range(_QB // 16):
      sl = pl.ds(h * 16, 16)
      uvecs.append(iv_v[0, sl])
      dvecs.append(iv_v[1, sl])
      svecs.append(iv_v[2, sl])
    ubases = [lax.shift_right_logical(v, 3) * 8 for v in uvecs]
    dbases = [lax.shift_right_logical(v, 3) * 8 for v in dvecs]
    sbases = [lax.shift_right_logical(v, 3) * 8 for v in svecs]
    for s in range(_QB):
      ub = pl.multiple_of(ubases[s // 16][s % 16], 8)
      gcs.append(pltpu.async_copy(utab.at[pl.ds(ub, 8)], tbuf.at[s], sG))
    for s in range(_QB):
      db = pl.multiple_of(dbases[s // 16][s % 16], 8)
      gcs.append(pltpu.async_copy(dtab.at[pl.ds(db, 8)], dbuf.at[s], sG))
    for s in range(_QB):
      sb2 = pl.multiple_of(sbases[s // 16][s % 16], 8)
      gcs.append(pltpu.async_copy(stab.at[pl.ds(sb2, 8)], sbuf.at[s], sG))
    for cp in gcs:
      cp.wait()
    ursels = [lax.bitwise_and(v, 7) for v in uvecs]
    drsels = [lax.bitwise_and(v, 7) for v in dvecs]
    srsels = [lax.bitwise_and(v, 7) for v in svecs]
    for s in range(_QB):
      ur = ursels[s // 16][s % 16]
      dr = drsels[s // 16][s % 16]
      sr = srsels[s // 16][s % 16]
      for k in range(4):
        sl = pl.ds(k * 16, 16)
        cb1[s, sl] = tbuf[s, ur, sl]
      for k in range(4):
        sl = pl.ds(k * 16, 16)
        cb1[s, pl.ds(64 + k * 16, 16)] = dbuf[s, dr, sl]
      for k in range(2):
        sl = pl.ds(k * 16, 16)
        cb2[s, sl] = sbuf[s, sr, sl]
    o1 = pltpu.async_copy(cb1, out1.at[pl.ds(b, _QB)], sO)
    o2 = pltpu.async_copy(cb2, out2.at[pl.ds(b, _QB)], sO)
    o1.wait()
    o2.wait()
    return carry

  lax.fori_loop(0, _NQ, chunk, 0)


@functools.cache
def _sc_gather_kernel():
  f32 = jnp.float32
  i32 = jnp.int32
  return pl.kernel(
      _sc_body,
      out_type=[
          jax.ShapeDtypeStruct((_B, 128), f32),
          jax.ShapeDtypeStruct((_B, 128), f32),
      ],
      mesh=plsc.VectorSubcoreMesh(core_axis_name="c", subcore_axis_name="s",
                                  num_cores=_NC, num_subcores=_NS),
      scratch_types=[
          pltpu.VMEM((3, _QB), i32),
          pltpu.VMEM((_QB, 8, 64), f32),
          pltpu.VMEM((_QB, 8, 64), f32),
          pltpu.VMEM((_QB, 8, 32), f32),
          pltpu.VMEM((_QB, 128), f32),
          pltpu.VMEM((_QB, 128), f32),
          pltpu.SemaphoreType.DMA,
          pltpu.SemaphoreType.DMA,
          pltpu.SemaphoreType.DMA,
      ],
      compiler_params=pltpu.CompilerParams(use_tc_tiling_on_sc=True,
                                           needs_layout_passes=False),
  )


def _tc_body(ud, sp, pools,
             age, gender, uloc, utime, uday, rec,
             tags, tastes, price, order, rating, iloc, itime, iday,
             age_W, age_b, gender_tab, uloc_W, uloc_b, utime_W, utime_b,
             uday_tab, rec_W, rec_b,
             price_W, price_b, order_W, order_b, rating_W, rating_b,
             iloc_W, iloc_b, itime_W, itime_b, iday_tab,
             up_W, up_b, ip_W, ip_b,
             un_out, it_out, sc_out):
  f32 = jnp.float32
  Wu = up_W[...]
  Wi = ip_W[...]
  udv = ud[...]
  spv = sp[...]

  uv = jnp.dot(udv[:, 0:64], Wu[0:64], preferred_element_type=f32)
  uv += age[...] * jnp.dot(age_W[...], Wu[64:80], preferred_element_type=f32)
  g1h = (gender[...] == lax.broadcasted_iota(jnp.int32, (_BC, 3), 1)).astype(f32)
  uv += jnp.dot(g1h, jnp.dot(gender_tab[...], Wu[80:96],
                             preferred_element_type=f32),
                preferred_element_type=f32)
  uv += jnp.dot(uloc[...], jnp.dot(uloc_W[...], Wu[96:112],
                                   preferred_element_type=f32),
                preferred_element_type=f32)
  uv += utime[...] * jnp.dot(utime_W[...], Wu[112:120], preferred_element_type=f32)
  ud1h = (uday[...] == lax.broadcasted_iota(jnp.int32, (_BC, 7), 1)).astype(f32)
  uv += jnp.dot(ud1h, jnp.dot(uday_tab[...], Wu[120:128],
                              preferred_element_type=f32),
                preferred_element_type=f32)
  uv += rec[...] * jnp.dot(rec_W[...], Wu[128:144], preferred_element_type=f32)
  ubias = (jnp.dot(age_b[...], Wu[64:80], preferred_element_type=f32)
           + jnp.dot(uloc_b[...], Wu[96:112], preferred_element_type=f32)
           + jnp.dot(utime_b[...], Wu[112:120], preferred_element_type=f32)
           + jnp.dot(rec_b[...], Wu[128:144], preferred_element_type=f32)
           + up_b[...])
  uv += ubias

  iv = jnp.dot(udv[:, 64:128], Wi[0:64], preferred_element_type=f32)
  iv += jnp.dot(spv[:, 0:32], Wi[64:96], preferred_element_type=f32)
  m_tag = (tags[...] != 0).astype(f32)
  inv_t = 1.0 / (jnp.sum(m_tag, axis=1, keepdims=True) + 1e-08)
  pk = pools[...]
  iv += jnp.dot(pk[:, 0:16] * inv_t, Wi[96:112], preferred_element_type=f32)
  m_ts = (tastes[...] != 0).astype(f32)
  inv_s = 1.0 / (jnp.sum(m_ts, axis=1, keepdims=True) + 1e-08)
  iv += jnp.dot(pk[:, 16:32] * inv_s, Wi[112:128], preferred_element_type=f32)
  iv += jnp.dot(pk[:, 32:48], Wi[128:144], preferred_element_type=f32)
  iv += price[...] * jnp.dot(price_W[...], Wi[144:160], preferred_element_type=f32)
  iv += order[...] * jnp.dot(order_W[...], Wi[160:168], preferred_element_type=f32)
  iv += rating[...] * jnp.dot(rating_W[...], Wi[168:176], preferred_element_type=f32)
  iv += jnp.dot(iloc[...], jnp.dot(iloc_W[...], Wi[176:192],
                                   preferred_element_type=f32),
                preferred_element_type=f32)
  iv += itime[...] * jnp.dot(itime_W[...], Wi[192:200], preferred_element_type=f32)
  id1h = (iday[...] == lax.broadcasted_iota(jnp.int32, (_BC, 7), 1)).astype(f32)
  iv += jnp.dot(id1h, jnp.dot(iday_tab[...], Wi[200:208],
                              preferred_element_type=f32),
                preferred_element_type=f32)
  ibias = (jnp.dot(price_b[...], Wi[144:160], preferred_element_type=f32)
           + jnp.dot(order_b[...], Wi[160:168], preferred_element_type=f32)
           + jnp.dot(rating_b[...], Wi[168:176], preferred_element_type=f32)
           + jnp.dot(iloc_b[...], Wi[176:192], preferred_element_type=f32)
           + jnp.dot(itime_b[...], Wi[192:200], preferred_element_type=f32)
           + ip_b[...])
  iv += ibias

  un = uv / jnp.maximum(jnp.sqrt(jnp.sum(uv * uv, axis=-1, keepdims=True)), 1e-12)
  it = iv / jnp.maximum(jnp.sqrt(jnp.sum(iv * iv, axis=-1, keepdims=True)), 1e-12)
  un_out[...] = un
  it_out[...] = it
  sc_out[...] = jnp.sum(un * it, axis=-1, keepdims=True)


def _chunk(d):
  return pl.BlockSpec((_BC, d), lambda i: (i, 0))


def _full(shape):
  return pl.BlockSpec(shape, lambda i: (0,) * len(shape))


def kernel(user_id, age, gender, user_location, user_time_of_day,
           user_day_of_week, recency, dish_id, store_id, tags, tastes,
           category, price, order_times, rating, item_location,
           item_time_of_day, item_day_of_week, user_emb_table, user_age_W,
           user_age_b, user_gender_table, user_location_W, user_location_b,
           user_time_W, user_time_b, user_day_table, user_recency_W,
           user_recency_b, dish_emb_table, store_emb_table, tag_emb_table,
           taste_emb_table, cat_emb_table, dish_price_W, dish_price_b,
           dish_order_times_W, dish_order_times_b, dish_rating_W,
           dish_rating_b, dish_location_W, dish_location_b, dish_time_W,
           dish_time_b, dish_day_table, user_proj_W, user_proj_b,
           item_proj_W, item_proj_b):
  i32 = jnp.int32
  uid1 = user_id.astype(i32)
  did1 = dish_id.astype(i32)
  sid1 = store_id.astype(i32)
  tag1 = tags.astype(i32).reshape(_B * 5)
  tas1 = tastes.astype(i32).reshape(_B * 3)
  cat1 = category.astype(i32)

  t1 = tag_emb_table.reshape(-1)
  ts1 = taste_emb_table.reshape(-1)
  c1 = cat_emb_table.reshape(-1)

  pools = _sc_pool_kernel()(t1, tag1, ts1, tas1, c1, cat1)[0]
  ud, sp = _sc_gather_kernel()(
      user_emb_table, uid1, dish_emb_table, did1, store_emb_table, sid1)

  grid = (_B // _BC,)
  un, it, sc = pl.pallas_call(
      _tc_body,
      grid=grid,
      in_specs=[
          _chunk(128), _chunk(128), _chunk(48),
          _chunk(1), _chunk(1), _chunk(2), _chunk(1), _chunk(1), _chunk(1),
          _chunk(5), _chunk(3), _chunk(1), _chunk(1), _chunk(1), _chunk(2),
          _chunk(1), _chunk(1),
          _full((1, 16)), _full((1, 16)), _full((3, 16)), _full((2, 16)),
          _full((1, 16)), _full((1, 8)), _full((1, 8)), _full((7, 8)),
          _full((1, 16)), _full((1, 16)),
          _full((1, 16)), _full((1, 16)), _full((1, 8)), _full((1, 8)),
          _full((1, 8)), _full((1, 8)), _full((2, 16)), _full((1, 16)),
          _full((1, 8)), _full((1, 8)), _full((7, 8)),
          _full((144, 64)), _full((1, 64)), _full((208, 64)), _full((1, 64)),
      ],
      out_specs=[_chunk(64), _chunk(64), _chunk(1)],
      out_shape=[
          jax.ShapeDtypeStruct((_B, 64), jnp.float32),
          jax.ShapeDtypeStruct((_B, 64), jnp.float32),
          jax.ShapeDtypeStruct((_B, 1), jnp.float32),
      ],
  )(
      ud, sp, pools,
      age.reshape(_B, 1), gender.astype(i32).reshape(_B, 1), user_location,
      user_time_of_day.reshape(_B, 1),
      user_day_of_week.astype(i32).reshape(_B, 1), recency.reshape(_B, 1),
      tags.astype(i32), tastes.astype(i32), price.reshape(_B, 1),
      order_times.reshape(_B, 1), rating.reshape(_B, 1), item_location,
      item_time_of_day.reshape(_B, 1),
      item_day_of_week.astype(i32).reshape(_B, 1),
      user_age_W, user_age_b.reshape(1, 16), user_gender_table,
      user_location_W, user_location_b.reshape(1, 16), user_time_W,
      user_time_b.reshape(1, 8), user_day_table, user_recency_W,
      user_recency_b.reshape(1, 16),
      dish_price_W, dish_price_b.reshape(1, 16), dish_order_times_W,
      dish_order_times_b.reshape(1, 8), dish_rating_W,
      dish_rating_b.reshape(1, 8), dish_location_W,
      dish_location_b.reshape(1, 16), dish_time_W, dish_time_b.reshape(1, 8),
      dish_day_table,
      user_proj_W, user_proj_b.reshape(1, 64), item_proj_W,
      item_proj_b.reshape(1, 64),
  )
  return (un, it, sc.reshape(_B))


# speedup vs baseline: 1.3830x; 1.0343x over previous
import functools

import jax
import jax.numpy as jnp
from jax import lax
from jax.experimental import pallas as pl
from jax.experimental.pallas import tpu as pltpu
from jax.experimental.pallas import tpu_sc as plsc

_B = 16384
_NC = 2
_NS = 16
_NW = _NC * _NS
_PW = _B // _NW
_QB = 32
_NQ = _PW // _QB

_BC = 1024


def _pool_group(idbuf, tabv, dstv, sb, slots, dofs):
  i32 = jnp.int32
  f32 = jnp.float32
  iota = lax.iota(i32, 16)
  accs = [jnp.zeros((16,), f32) for _ in range(16)]
  for j in range(slots):
    pvec = iota * slots + (sb * slots + j)
    tv = plsc.load_gather(idbuf, [pvec])
    mf = (tv != 0).astype(f32)
    fbase = tv * 16
    for d in range(16):
      vals = plsc.load_gather(tabv, [fbase + d])
      accs[d] = accs[d] + vals * mf
  rowv = sb + iota
  for d in range(16):
    plsc.store_scatter(dstv, [rowv, jnp.full((16,), dofs + d, i32)], accs[d])


def _lookup_group(idbuf, tabv, dstv, sb, dofs):
  i32 = jnp.int32
  iota = lax.iota(i32, 16)
  cv = idbuf[pl.ds(sb, 16)]
  fbase = cv * 16
  rowv = sb + iota
  for d in range(16):
    vals = plsc.load_gather(tabv, [fbase + d])
    plsc.store_scatter(dstv, [rowv, jnp.full((16,), dofs + d, i32)], vals)


def _sc_pool_body(ttab, tagid, tstab, tasteid, ctab, catid,
                  pools,
                  tagtab_v, tastetab_v, cattab_v, it1, its1, ict1, psum,
                  sT, sO):
  wid = lax.axis_index("s") * _NC + lax.axis_index("c")
  b0 = wid * _PW
  pre = [
      pltpu.async_copy(ttab, tagtab_v, sT),
      pltpu.async_copy(tstab, tastetab_v, sT),
      pltpu.async_copy(ctab, cattab_v, sT),
      pltpu.async_copy(tagid.at[pl.ds(b0 * 5, _PW * 5)], it1, sT),
      pltpu.async_copy(tasteid.at[pl.ds(b0 * 3, _PW * 3)], its1, sT),
      pltpu.async_copy(catid.at[pl.ds(b0, _PW)], ict1, sT),
  ]
  for cp in pre:
    cp.wait()

  def grp(g, carry):
    sb = g * 16
    _pool_group(it1, tagtab_v, psum, sb, 5, 0)
    _pool_group(its1, tastetab_v, psum, sb, 3, 16)
    _lookup_group(ict1, cattab_v, psum, sb, 32)
    return carry

  lax.fori_loop(0, _PW // 16, grp, 0)
  pltpu.async_copy(psum, pools.at[pl.ds(b0, _PW)], sO).wait()


@functools.cache
def _sc_pool_kernel():
  f32 = jnp.float32
  i32 = jnp.int32
  return pl.kernel(
      _sc_pool_body,
      out_type=[jax.ShapeDtypeStruct((_B, 48), f32)],
      mesh=plsc.VectorSubcoreMesh(core_axis_name="c", subcore_axis_name="s",
                                  num_cores=_NC, num_subcores=_NS),
      scratch_types=[
          pltpu.VMEM((16000,), f32),
          pltpu.VMEM((16000,), f32),
          pltpu.VMEM((16000,), f32),
          pltpu.VMEM((_PW * 5,), i32),
          pltpu.VMEM((_PW * 3,), i32),
          pltpu.VMEM((_PW,), i32),
          pltpu.VMEM((_PW, 48), f32),
          pltpu.SemaphoreType.DMA,
          pltpu.SemaphoreType.DMA,
      ],
      compiler_params=pltpu.CompilerParams(use_tc_tiling_on_sc=False,
                                           needs_layout_passes=False),
  )


def _sc_user_body(utab, uid, out1, iv_v, tbuf, cb1, sI, sG, sO):
  wid = lax.axis_index("s") * _NC + lax.axis_index("c")
  b0 = wid * _PW

  def chunk(q, carry):
    b = b0 + q * _QB
    pltpu.async_copy(uid.at[pl.ds(b, _QB)], iv_v.at[0], sI).wait()
    uvecs = [iv_v[0, pl.ds(h * 16, 16)] for h in range(_QB // 16)]
    ubases = [lax.shift_right_logical(v, 3) * 8 for v in uvecs]
    gcs = []
    for s in range(_QB):
      ub = pl.multiple_of(ubases[s // 16][s % 16], 8)
      gcs.append(pltpu.async_copy(utab.at[pl.ds(ub, 8)], tbuf.at[s], sG))
    for cp in gcs:
      cp.wait()
    ursels = [lax.bitwise_and(v, 7) for v in uvecs]
    for s in range(_QB):
      ur = ursels[s // 16][s % 16]
      for k in range(4):
        sl = pl.ds(k * 16, 16)
        cb1[s, sl] = tbuf[s, ur, sl]
    pltpu.async_copy(cb1, out1.at[pl.ds(b, _QB)], sO).wait()
    return carry

  lax.fori_loop(0, _NQ, chunk, 0)


def _sc_ds_body(dtab, did, stab, sid, out2, iv_v, dbuf, sbuf, cb2,
                sI, sG, sO):
  wid = lax.axis_index("s") * _NC + lax.axis_index("c")
  b0 = wid * _PW

  def chunk(q, carry):
    b = b0 + q * _QB
    ics = [
        pltpu.async_copy(did.at[pl.ds(b, _QB)], iv_v.at[0], sI),
        pltpu.async_copy(sid.at[pl.ds(b, _QB)], iv_v.at[1], sI),
    ]
    for cp in ics:
      cp.wait()
    dvecs = [iv_v[0, pl.ds(h * 16, 16)] for h in range(_QB // 16)]
    svecs = [iv_v[1, pl.ds(h * 16, 16)] for h in range(_QB // 16)]
    dbases = [lax.shift_right_logical(v, 3) * 8 for v in dvecs]
    sbases = [lax.shift_right_logical(v, 3) * 8 for v in svecs]
    gcs = []
    for s in range(_QB):
      db = pl.multiple_of(dbases[s // 16][s % 16], 8)
      gcs.append(pltpu.async_copy(dtab.at[pl.ds(db, 8)], dbuf.at[s], sG))
    for s in range(_QB):
      sb2 = pl.multiple_of(sbases[s // 16][s % 16], 8)
      gcs.append(pltpu.async_copy(stab.at[pl.ds(sb2, 8)], sbuf.at[s], sG))
    for cp in gcs:
      cp.wait()
    drsels = [lax.bitwise_and(v, 7) for v in dvecs]
    srsels = [lax.bitwise_and(v, 7) for v in svecs]
    for s in range(_QB):
      dr = drsels[s // 16][s % 16]
      sr = srsels[s // 16][s % 16]
      for k in range(4):
        sl = pl.ds(k * 16, 16)
        cb2[s, sl] = dbuf[s, dr, sl]
      for k in range(2):
        sl = pl.ds(k * 16, 16)
        cb2[s, pl.ds(64 + k * 16, 16)] = sbuf[s, sr, sl]
    pltpu.async_copy(cb2, out2.at[pl.ds(b, _QB)], sO).wait()
    return carry

  lax.fori_loop(0, _NQ, chunk, 0)


@functools.cache
def _sc_user_kernel():
  f32 = jnp.float32
  i32 = jnp.int32
  return pl.kernel(
      _sc_user_body,
      out_type=[jax.ShapeDtypeStruct((_B, 64), f32)],
      mesh=plsc.VectorSubcoreMesh(core_axis_name="c", subcore_axis_name="s",
                                  num_cores=_NC, num_subcores=_NS),
      scratch_types=[
          pltpu.VMEM((1, _QB), i32),
          pltpu.VMEM((_QB, 8, 64), f32),
          pltpu.VMEM((_QB, 64), f32),
          pltpu.SemaphoreType.DMA,
          pltpu.SemaphoreType.DMA,
          pltpu.SemaphoreType.DMA,
      ],
      compiler_params=pltpu.CompilerParams(use_tc_tiling_on_sc=True,
                                           needs_layout_passes=False),
  )


@functools.cache
def _sc_ds_kernel():
  f32 = jnp.float32
  i32 = jnp.int32
  return pl.kernel(
      _sc_ds_body,
      out_type=[jax.ShapeDtypeStruct((_B, 128), f32)],
      mesh=plsc.VectorSubcoreMesh(core_axis_name="c", subcore_axis_name="s",
                                  num_cores=_NC, num_subcores=_NS),
      scratch_types=[
          pltpu.VMEM((2, _QB), i32),
          pltpu.VMEM((_QB, 8, 64), f32),
          pltpu.VMEM((_QB, 8, 32), f32),
          pltpu.VMEM((_QB, 128), f32),
          pltpu.SemaphoreType.DMA,
          pltpu.SemaphoreType.DMA,
          pltpu.SemaphoreType.DMA,
      ],
      compiler_params=pltpu.CompilerParams(use_tc_tiling_on_sc=True,
                                           needs_layout_passes=False),
  )


def _tc_body(ud, sp, pools,
             age, gender, uloc, utime, uday, rec,
             tags, tastes, price, order, rating, iloc, itime, iday,
             age_W, age_b, gender_tab, uloc_W, uloc_b, utime_W, utime_b,
             uday_tab, rec_W, rec_b,
             price_W, price_b, order_W, order_b, rating_W, rating_b,
             iloc_W, iloc_b, itime_W, itime_b, iday_tab,
             up_W, up_b, ip_W, ip_b,
             un_out, it_out, sc_out):
  f32 = jnp.float32
  Wu = up_W[...]
  Wi = ip_W[...]
  udv = ud[...]
  spv = sp[...]

  uv = jnp.dot(udv, Wu[0:64], preferred_element_type=f32)
  uv += age[...] * jnp.dot(age_W[...], Wu[64:80], preferred_element_type=f32)
  g1h = (gender[...] == lax.broadcasted_iota(jnp.int32, (_BC, 3), 1)).astype(f32)
  uv += jnp.dot(g1h, jnp.dot(gender_tab[...], Wu[80:96],
                             preferred_element_type=f32),
                preferred_element_type=f32)
  uv += jnp.dot(uloc[...], jnp.dot(uloc_W[...], Wu[96:112],
                                   preferred_element_type=f32),
                preferred_element_type=f32)
  uv += utime[...] * jnp.dot(utime_W[...], Wu[112:120], preferred_element_type=f32)
  ud1h = (uday[...] == lax.broadcasted_iota(jnp.int32, (_BC, 7), 1)).astype(f32)
  uv += jnp.dot(ud1h, jnp.dot(uday_tab[...], Wu[120:128],
                              preferred_element_type=f32),
                preferred_element_type=f32)
  uv += rec[...] * jnp.dot(rec_W[...], Wu[128:144], preferred_element_type=f32)
  ubias = (jnp.dot(age_b[...], Wu[64:80], preferred_element_type=f32)
           + jnp.dot(uloc_b[...], Wu[96:112], preferred_element_type=f32)
           + jnp.dot(utime_b[...], Wu[112:120], preferred_element_type=f32)
           + jnp.dot(rec_b[...], Wu[128:144], preferred_element_type=f32)
           + up_b[...])
  uv += ubias

  iv = jnp.dot(spv[:, 0:64], Wi[0:64], preferred_element_type=f32)
  iv += jnp.dot(spv[:, 64:96], Wi[64:96], preferred_element_type=f32)
  m_tag = (tags[...] != 0).astype(f32)
  inv_t = 1.0 / (jnp.sum(m_tag, axis=1, keepdims=True) + 1e-08)
  pk = pools[...]
  iv += jnp.dot(pk[:, 0:16] * inv_t, Wi[96:112], preferred_element_type=f32)
  m_ts = (tastes[...] != 0).astype(f32)
  inv_s = 1.0 / (jnp.sum(m_ts, axis=1, keepdims=True) + 1e-08)
  iv += jnp.dot(pk[:, 16:32] * inv_s, Wi[112:128], preferred_element_type=f32)
  iv += jnp.dot(pk[:, 32:48], Wi[128:144], preferred_element_type=f32)
  iv += price[...] * jnp.dot(price_W[...], Wi[144:160], preferred_element_type=f32)
  iv += order[...] * jnp.dot(order_W[...], Wi[160:168], preferred_element_type=f32)
  iv += rating[...] * jnp.dot(rating_W[...], Wi[168:176], preferred_element_type=f32)
  iv += jnp.dot(iloc[...], jnp.dot(iloc_W[...], Wi[176:192],
                                   preferred_element_type=f32),
                preferred_element_type=f32)
  iv += itime[...] * jnp.dot(itime_W[...], Wi[192:200], preferred_element_type=f32)
  id1h = (iday[...] == lax.broadcasted_iota(jnp.int32, (_BC, 7), 1)).astype(f32)
  iv += jnp.dot(id1h, jnp.dot(iday_tab[...], Wi[200:208],
                              preferred_element_type=f32),
                preferred_element_type=f32)
  ibias = (jnp.dot(price_b[...], Wi[144:160], preferred_element_type=f32)
           + jnp.dot(order_b[...], Wi[160:168], preferred_element_type=f32)
           + jnp.dot(rating_b[...], Wi[168:176], preferred_element_type=f32)
           + jnp.dot(iloc_b[...], Wi[176:192], preferred_element_type=f32)
           + jnp.dot(itime_b[...], Wi[192:200], preferred_element_type=f32)
           + ip_b[...])
  iv += ibias

  un = uv / jnp.maximum(jnp.sqrt(jnp.sum(uv * uv, axis=-1, keepdims=True)), 1e-12)
  it = iv / jnp.maximum(jnp.sqrt(jnp.sum(iv * iv, axis=-1, keepdims=True)), 1e-12)
  un_out[...] = un
  it_out[...] = it
  sc_out[...] = jnp.sum(un * it, axis=-1, keepdims=True)


def _chunk(d):
  return pl.BlockSpec((_BC, d), lambda i: (i, 0))


def _full(shape):
  return pl.BlockSpec(shape, lambda i: (0,) * len(shape))


def kernel(user_id, age, gender, user_location, user_time_of_day,
           user_day_of_week, recency, dish_id, store_id, tags, tastes,
           category, price, order_times, rating, item_location,
           item_time_of_day, item_day_of_week, user_emb_table, user_age_W,
           user_age_b, user_gender_table, user_location_W, user_location_b,
           user_time_W, user_time_b, user_day_table, user_recency_W,
           user_recency_b, dish_emb_table, store_emb_table, tag_emb_table,
           taste_emb_table, cat_emb_table, dish_price_W, dish_price_b,
           dish_order_times_W, dish_order_times_b, dish_rating_W,
           dish_rating_b, dish_location_W, dish_location_b, dish_time_W,
           dish_time_b, dish_day_table, user_proj_W, user_proj_b,
           item_proj_W, item_proj_b):
  i32 = jnp.int32
  uid1 = user_id.astype(i32)
  did1 = dish_id.astype(i32)
  sid1 = store_id.astype(i32)
  tag1 = tags.astype(i32).reshape(_B * 5)
  tas1 = tastes.astype(i32).reshape(_B * 3)
  cat1 = category.astype(i32)

  t1 = tag_emb_table.reshape(-1)
  ts1 = taste_emb_table.reshape(-1)
  c1 = cat_emb_table.reshape(-1)

  pools = _sc_pool_kernel()(t1, tag1, ts1, tas1, c1, cat1)[0]
  sp = _sc_ds_kernel()(dish_emb_table, did1, store_emb_table, sid1)[0]
  ud = _sc_user_kernel()(user_emb_table, uid1)[0]

  grid = (_B // _BC,)
  un, it, sc = pl.pallas_call(
      _tc_body,
      grid=grid,
      in_specs=[
          _chunk(64), _chunk(128), _chunk(48),
          _chunk(1), _chunk(1), _chunk(2), _chunk(1), _chunk(1), _chunk(1),
          _chunk(5), _chunk(3), _chunk(1), _chunk(1), _chunk(1), _chunk(2),
          _chunk(1), _chunk(1),
          _full((1, 16)), _full((1, 16)), _full((3, 16)), _full((2, 16)),
          _full((1, 16)), _full((1, 8)), _full((1, 8)), _full((7, 8)),
          _full((1, 16)), _full((1, 16)),
          _full((1, 16)), _full((1, 16)), _full((1, 8)), _full((1, 8)),
          _full((1, 8)), _full((1, 8)), _full((2, 16)), _full((1, 16)),
          _full((1, 8)), _full((1, 8)), _full((7, 8)),
          _full((144, 64)), _full((1, 64)), _full((208, 64)), _full((1, 64)),
      ],
      out_specs=[_chunk(64), _chunk(64), _chunk(1)],
      out_shape=[
          jax.ShapeDtypeStruct((_B, 64), jnp.float32),
          jax.ShapeDtypeStruct((_B, 64), jnp.float32),
          jax.ShapeDtypeStruct((_B, 1), jnp.float32),
      ],
  )(
      ud, sp, pools,
      age.reshape(_B, 1), gender.astype(i32).reshape(_B, 1), user_location,
      user_time_of_day.reshape(_B, 1),
      user_day_of_week.astype(i32).reshape(_B, 1), recency.reshape(_B, 1),
      tags.astype(i32), tastes.astype(i32), price.reshape(_B, 1),
      order_times.reshape(_B, 1), rating.reshape(_B, 1), item_location,
      item_time_of_day.reshape(_B, 1),
      item_day_of_week.astype(i32).reshape(_B, 1),
      user_age_W, user_age_b.reshape(1, 16), user_gender_table,
      user_location_W, user_location_b.reshape(1, 16), user_time_W,
      user_time_b.reshape(1, 8), user_day_table, user_recency_W,
      user_recency_b.reshape(1, 16),
      dish_price_W, dish_price_b.reshape(1, 16), dish_order_times_W,
      dish_order_times_b.reshape(1, 8), dish_rating_W,
      dish_rating_b.reshape(1, 8), dish_location_W,
      dish_location_b.reshape(1, 16), dish_time_W, dish_time_b.reshape(1, 8),
      dish_day_table,
      user_proj_W, user_proj_b.reshape(1, 64), item_proj_W,
      item_proj_b.reshape(1, 64),
  )
  return (un, it, sc.reshape(_B))
